# Initial kernel scaffold; baseline (speedup 1.0000x reference)
import jax
import jax.numpy as jnp
from jax.experimental import pallas as pl


def kernel(x, edge_index, params):
    raise NotImplementedError("write your pallas kernel here")


# speedup vs baseline: 7.4327x; 7.4327x over previous
import functools

import jax
import jax.numpy as jnp
from jax import lax
from jax.experimental import pallas as pl
from jax.experimental.pallas import tpu as pltpu
from jax.experimental.pallas import tpu_sc as plsc

_NC = 2
_NS = 16
_NW = _NC * _NS
_CH = 80


def _ln(h, w, b):
    mu = jnp.mean(h, axis=-1, keepdims=True)
    d = h - mu
    var = jnp.mean(d * d, axis=-1, keepdims=True)
    return d / jnp.sqrt(var + 1e-5) * w + b


def _l2n(h):
    nrm = jnp.sqrt(jnp.sum(h * h, axis=-1, keepdims=True))
    return h / jnp.maximum(nrm, 1e-12)


def _stage_a_body(x_ref, bnw_ref, bnb_ref, fw_ref, w1_ref, b1_ref, l1w_ref,
                  l1b_ref, w2_ref, b2_ref, l2w_ref, l2b_ref, wl0_ref, wr0_ref,
                  d_ref, p0_ref, r0_ref):
    x = x_ref[...]
    mu = jnp.mean(x, axis=0, keepdims=True)
    xc = x - mu
    var = jnp.mean(xc * xc, axis=0, keepdims=True)
    h = xc * (bnw_ref[...] / jnp.sqrt(var + 1e-5)) + bnb_ref[...]
    xw = h * (1.0 + 0.5 * jax.nn.sigmoid(fw_ref[...]))
    t = jnp.dot(xw, w1_ref[...], preferred_element_type=jnp.float32) + b1_ref[...]
    t = jnp.maximum(_ln(t, l1w_ref[...], l1b_ref[...]), 0.0)
    t = jnp.dot(t, w2_ref[...], preferred_element_type=jnp.float32) + b2_ref[...]
    d_ref[...] = jnp.maximum(_ln(t, l2w_ref[...], l2b_ref[...]), 0.0)
    p = jnp.dot(xw, wl0_ref[...], preferred_element_type=jnp.float32)
    col = lax.broadcasted_iota(jnp.int32, (x.shape[0], 16), 1)
    ones_col = jnp.where(col == 0, 1.0, 0.0)
    p0_ref[...] = jnp.concatenate([p, ones_col], axis=1)
    r0_ref[...] = jnp.dot(xw, wr0_ref[...], preferred_element_type=jnp.float32)


def _stage_a(x, bnw, bnb, fw, w1, b1, l1w, l1b, w2, b2, l2w, l2b, wl0, wr0):
    n = x.shape[0]
    return pl.pallas_call(
        _stage_a_body,
        out_shape=[
            jax.ShapeDtypeStruct((n, 64), jnp.float32),
            jax.ShapeDtypeStruct((n, 80), jnp.float32),
            jax.ShapeDtypeStruct((n, 64), jnp.float32),
        ],
    )(x, bnw, bnb, fw, w1, b1, l1w, l1b, w2, b2, l2w, l2b, wl0, wr0)


def _stage_b_body(agg_ref, r0_ref, nw_ref, nb_ref, wl_ref, wr_ref,
                  g_ref, p_ref, r_ref, inv_ref):
    n = r0_ref.shape[0]
    agg = agg_ref[0, :n] + agg_ref[1, :n]
    deg = jnp.sum(agg[:, 64:80], axis=-1, keepdims=True)
    inv = 1.0 / jnp.maximum(deg, 1.0)
    sage = agg[:, :64] * inv + r0_ref[...]
    g = jnp.maximum(_ln(_l2n(sage), nw_ref[...], nb_ref[...]), 0.0)
    g_ref[...] = g
    p_ref[...] = jnp.dot(g, wl_ref[...], preferred_element_type=jnp.float32)
    r_ref[...] = jnp.dot(g, wr_ref[...], preferred_element_type=jnp.float32)
    inv_ref[...] = jnp.broadcast_to(inv, (agg.shape[0], 64))


def _stage_b(agg0, r0, nw, nb, wl, wr):
    n = r0.shape[0]
    s64 = jax.ShapeDtypeStruct((n, 64), jnp.float32)
    return pl.pallas_call(
        _stage_b_body, out_shape=[s64, s64, s64, s64],
    )(agg0, r0, nw, nb, wl, wr)


def _stage_c_body(agg_ref, rin_ref, gin_ref, inv_ref, nw_ref, nb_ref,
                  wl_ref, wr_ref, g_ref, p_ref, r_ref):
    n = rin_ref.shape[0]
    agg = agg_ref[0, :n] + agg_ref[1, :n]
    sage = agg * inv_ref[...] + rin_ref[...]
    g = jnp.maximum(_ln(_l2n(sage), nw_ref[...], nb_ref[...]), 0.0)
    g = g + 0.1 * gin_ref[...]
    g_ref[...] = g
    p_ref[...] = jnp.dot(g, wl_ref[...], preferred_element_type=jnp.float32)
    r_ref[...] = jnp.dot(g, wr_ref[...], preferred_element_type=jnp.float32)


def _stage_c(agg, rin, gin, inv, nw, nb, wl, wr):
    n = rin.shape[0]
    s64 = jax.ShapeDtypeStruct((n, 64), jnp.float32)
    return pl.pallas_call(
        _stage_c_body, out_shape=[s64, s64, s64],
    )(agg, rin, gin, inv, nw, nb, wl, wr)


def _head(comb, w1, b1, lw, lb, w2, b2, w3, b3, scale, bias):
    o = jnp.dot(comb, w1, preferred_element_type=jnp.float32) + b1
    o = jnp.maximum(_ln(o, lw, lb), 0.0)
    o = jnp.maximum(jnp.dot(o, w2, preferred_element_type=jnp.float32) + b2, 0.0)
    o = jnp.dot(o, w3, preferred_element_type=jnp.float32) + b3
    return o * jnp.abs(scale) + bias


def _stage_d_body(agg_ref, rin_ref, gin_ref, inv_ref, d_ref, nw_ref, nb_ref,
                  hw_refs, out_ref):
    n = rin_ref.shape[0]
    agg = agg_ref[0, :n] + agg_ref[1, :n]
    sage = agg * inv_ref[...] + rin_ref[...]
    g = jnp.maximum(_ln(_l2n(sage), nw_ref[...], nb_ref[...]), 0.0)
    g = g + 0.1 * gin_ref[...]
    comb = jnp.concatenate([d_ref[...], g], axis=1)
    (rw1, rb1, rlw, rlb, rw2, rb2, rw3, rb3, rs, rbi,
     ew1, eb1, elw, elb, ew2, eb2, ew3, eb3, es, ebi) = [h[...] for h in hw_refs]
    rtt = _head(comb, rw1, rb1, rlw, rlb, rw2, rb2, rw3, rb3, rs, rbi)
    ret = _head(comb, ew1, eb1, elw, elb, ew2, eb2, ew3, eb3, es, ebi)
    col = lax.broadcasted_iota(jnp.int32, (comb.shape[0], 8), 1)
    out_ref[...] = jnp.where(col == 0, rtt, jnp.where(col == 1, ret, 0.0))


def _stage_d(agg, rin, gin, inv, d, nw, nb, head_ws):
    n = rin.shape[0]

    def body(agg_ref, rin_ref, gin_ref, inv_ref, d_ref, nw_ref, nb_ref, *hw):
        _stage_d_body(agg_ref, rin_ref, gin_ref, inv_ref, d_ref, nw_ref,
                      nb_ref, hw[:-1], hw[-1])

    return pl.pallas_call(
        body, out_shape=jax.ShapeDtypeStruct((n, 8), jnp.float32),
    )(agg, rin, gin, inv, d, nw, nb, *head_ws)


@functools.lru_cache(maxsize=None)
def _make_segsum(e_total, n_pad, w):
    epw = e_total // _NW
    nch = epw // _CH
    rps = n_pad // _NS
    mesh = plsc.VectorSubcoreMesh(core_axis_name="c", subcore_axis_name="s")

    @functools.partial(
        pl.kernel,
        out_type=jax.ShapeDtypeStruct((_NC, n_pad, w), jnp.float32),
        mesh=mesh,
        scratch_types=[
            pltpu.VMEM((nch, _CH), jnp.int32),
            pltpu.VMEM((nch, _CH), jnp.int32),
            pltpu.VMEM((_CH, w), jnp.float32),
            pltpu.VMEM_SHARED((n_pad, w), jnp.float32),
            pltpu.SemaphoreType.DMA,
        ],
        compiler_params=pltpu.CompilerParams(use_tc_tiling_on_sc=False),
    )
    def segsum(p_hbm, src_hbm, dst_hbm, z_hbm, out_hbm,
               src_v, dst_v, rows_v, agg_sh, sem):
        cid = lax.axis_index("c")
        sid = lax.axis_index("s")
        wid = sid * _NC + cid
        pltpu.sync_copy(z_hbm.at[pl.ds(sid * rps, rps)],
                        agg_sh.at[pl.ds(sid * rps, rps)])
        pltpu.sync_copy(src_hbm.at[wid], src_v)
        pltpu.sync_copy(dst_hbm.at[wid], dst_v)
        plsc.subcore_barrier()

        def body(i, carry):
            pltpu.async_copy(p_hbm.at[src_v.at[i]], rows_v, sem).wait()
            pltpu.sync_copy(rows_v, agg_sh.at[dst_v.at[i]], add=True)
            return carry

        lax.fori_loop(0, nch, body, 0)
        plsc.subcore_barrier()
        pltpu.sync_copy(agg_sh.at[pl.ds(sid * rps, rps)],
                        out_hbm.at[cid].at[pl.ds(sid * rps, rps)])

    return segsum


def kernel(x, edge_index, params):
    p = params
    n = x.shape[0]
    e_total = edge_index.shape[1]
    src3 = edge_index[0].reshape(_NW, e_total // _NW // _CH, _CH)
    dst3 = edge_index[1].reshape(_NW, e_total // _NW // _CH, _CH)

    row = lambda v: v.reshape(1, -1)
    d, p0, r0 = _stage_a(
        x, row(p['bn_w']), row(p['bn_b']), row(p['fw']),
        p['ft_w1'].T, row(p['ft_b1']), row(p['ft_ln1_w']), row(p['ft_ln1_b']),
        p['ft_w2'].T, row(p['ft_b2']), row(p['ft_ln2_w']), row(p['ft_ln2_b']),
        p['conv0_wl'].T, p['conv0_wr'].T)

    n_pad = ((n + 8 * _NS - 1) // (8 * _NS)) * (8 * _NS)
    z80 = jnp.zeros((n_pad, 80), jnp.float32)
    z64 = jnp.zeros((n_pad, 64), jnp.float32)
    segsum80 = _make_segsum(e_total, n_pad, 80)
    segsum64 = _make_segsum(e_total, n_pad, 64)

    agg0 = segsum80(p0, src3, dst3, z80)
    g1, p1, r1, inv = _stage_b(agg0, r0, row(p['norm0_w']), row(p['norm0_b']),
                               p['conv1_wl'].T, p['conv1_wr'].T)
    agg1 = segsum64(p1, src3, dst3, z64)
    g2, p2, r2 = _stage_c(agg1, r1, g1, inv, row(p['norm1_w']),
                          row(p['norm1_b']), p['conv2_wl'].T, p['conv2_wr'].T)
    agg2 = segsum64(p2, src3, dst3, z64)

    head_ws = []
    for pre in ['rtt', 'ret']:
        head_ws += [p[pre + '_w1'].T, row(p[pre + '_b1']),
                    row(p[pre + '_ln_w']), row(p[pre + '_ln_b']),
                    p[pre + '_w2'].T, row(p[pre + '_b2']),
                    p[pre + '_w3'].T, row(p[pre + '_b3']),
                    p[pre + '_scale'].reshape(1, 1), p[pre + '_bias'].reshape(1, 1)]
    o8 = _stage_d(agg2, r2, g2, inv, d, row(p['norm2_w']), row(p['norm2_b']),
                  head_ws)
    return o8[:, :2]


# speedup vs baseline: 8.7407x; 1.1760x over previous
import functools

import jax
import jax.numpy as jnp
from jax import lax
from jax.experimental import pallas as pl
from jax.experimental.pallas import tpu as pltpu
from jax.experimental.pallas import tpu_sc as plsc

_NC = 2
_NS = 16
_NW = _NC * _NS
_CH = 80


def _ln(h, w, b):
    mu = jnp.mean(h, axis=-1, keepdims=True)
    d = h - mu
    var = jnp.mean(d * d, axis=-1, keepdims=True)
    return d / jnp.sqrt(var + 1e-5) * w + b


def _l2n(h):
    nrm = jnp.sqrt(jnp.sum(h * h, axis=-1, keepdims=True))
    return h / jnp.maximum(nrm, 1e-12)


def _stage_a_body(x_ref, bnw_ref, bnb_ref, fw_ref, w1_ref, b1_ref, l1w_ref,
                  l1b_ref, w2_ref, b2_ref, l2w_ref, l2b_ref, wl0_ref, wr0_ref,
                  d_ref, p0_ref, r0_ref):
    x = x_ref[...]
    mu = jnp.mean(x, axis=0, keepdims=True)
    xc = x - mu
    var = jnp.mean(xc * xc, axis=0, keepdims=True)
    h = xc * (bnw_ref[...] / jnp.sqrt(var + 1e-5)) + bnb_ref[...]
    xw = h * (1.0 + 0.5 * jax.nn.sigmoid(fw_ref[...]))
    t = jnp.dot(xw, w1_ref[...], preferred_element_type=jnp.float32) + b1_ref[...]
    t = jnp.maximum(_ln(t, l1w_ref[...], l1b_ref[...]), 0.0)
    t = jnp.dot(t, w2_ref[...], preferred_element_type=jnp.float32) + b2_ref[...]
    d_ref[...] = jnp.maximum(_ln(t, l2w_ref[...], l2b_ref[...]), 0.0)
    p = jnp.dot(xw, wl0_ref[...], preferred_element_type=jnp.float32)
    col = lax.broadcasted_iota(jnp.int32, (x.shape[0], 16), 1)
    ones_col = jnp.where(col == 0, 1.0, 0.0)
    p0_ref[...] = jnp.concatenate([p, ones_col], axis=1)
    r0_ref[...] = jnp.dot(xw, wr0_ref[...], preferred_element_type=jnp.float32)


def _stage_a(x, bnw, bnb, fw, w1, b1, l1w, l1b, w2, b2, l2w, l2b, wl0, wr0):
    n = x.shape[0]
    return pl.pallas_call(
        _stage_a_body,
        out_shape=[
            jax.ShapeDtypeStruct((n, 64), jnp.float32),
            jax.ShapeDtypeStruct((n, 80), jnp.float32),
            jax.ShapeDtypeStruct((n, 64), jnp.float32),
        ],
    )(x, bnw, bnb, fw, w1, b1, l1w, l1b, w2, b2, l2w, l2b, wl0, wr0)


def _stage_b_body(agg_ref, r0_ref, nw_ref, nb_ref, wl_ref, wr_ref,
                  g_ref, p_ref, r_ref, inv_ref):
    n = r0_ref.shape[0]
    agg = agg_ref[0, :n] + agg_ref[1, :n]
    deg = jnp.sum(agg[:, 64:80], axis=-1, keepdims=True)
    inv = 1.0 / jnp.maximum(deg, 1.0)
    sage = agg[:, :64] * inv + r0_ref[...]
    g = jnp.maximum(_ln(_l2n(sage), nw_ref[...], nb_ref[...]), 0.0)
    g_ref[...] = g
    p_ref[...] = jnp.dot(g, wl_ref[...], preferred_element_type=jnp.float32)
    r_ref[...] = jnp.dot(g, wr_ref[...], preferred_element_type=jnp.float32)
    inv_ref[...] = jnp.broadcast_to(inv, (agg.shape[0], 64))


def _stage_b(agg0, r0, nw, nb, wl, wr):
    n = r0.shape[0]
    s64 = jax.ShapeDtypeStruct((n, 64), jnp.float32)
    return pl.pallas_call(
        _stage_b_body, out_shape=[s64, s64, s64, s64],
    )(agg0, r0, nw, nb, wl, wr)


def _stage_c_body(agg_ref, rin_ref, gin_ref, inv_ref, nw_ref, nb_ref,
                  wl_ref, wr_ref, g_ref, p_ref, r_ref):
    n = rin_ref.shape[0]
    agg = agg_ref[0, :n] + agg_ref[1, :n]
    sage = agg * inv_ref[...] + rin_ref[...]
    g = jnp.maximum(_ln(_l2n(sage), nw_ref[...], nb_ref[...]), 0.0)
    g = g + 0.1 * gin_ref[...]
    g_ref[...] = g
    p_ref[...] = jnp.dot(g, wl_ref[...], preferred_element_type=jnp.float32)
    r_ref[...] = jnp.dot(g, wr_ref[...], preferred_element_type=jnp.float32)


def _stage_c(agg, rin, gin, inv, nw, nb, wl, wr):
    n = rin.shape[0]
    s64 = jax.ShapeDtypeStruct((n, 64), jnp.float32)
    return pl.pallas_call(
        _stage_c_body, out_shape=[s64, s64, s64],
    )(agg, rin, gin, inv, nw, nb, wl, wr)


def _head(comb, w1, b1, lw, lb, w2, b2, w3, b3, scale, bias):
    o = jnp.dot(comb, w1, preferred_element_type=jnp.float32) + b1
    o = jnp.maximum(_ln(o, lw, lb), 0.0)
    o = jnp.maximum(jnp.dot(o, w2, preferred_element_type=jnp.float32) + b2, 0.0)
    o = jnp.dot(o, w3, preferred_element_type=jnp.float32) + b3
    return o * jnp.abs(scale) + bias


def _stage_d_body(agg_ref, rin_ref, gin_ref, inv_ref, d_ref, nw_ref, nb_ref,
                  hw_refs, out_ref):
    n = rin_ref.shape[0]
    agg = agg_ref[0, :n] + agg_ref[1, :n]
    sage = agg * inv_ref[...] + rin_ref[...]
    g = jnp.maximum(_ln(_l2n(sage), nw_ref[...], nb_ref[...]), 0.0)
    g = g + 0.1 * gin_ref[...]
    comb = jnp.concatenate([d_ref[...], g], axis=1)
    (rw1, rb1, rlw, rlb, rw2, rb2, rw3, rb3, rs, rbi,
     ew1, eb1, elw, elb, ew2, eb2, ew3, eb3, es, ebi) = [h[...] for h in hw_refs]
    rtt = _head(comb, rw1, rb1, rlw, rlb, rw2, rb2, rw3, rb3, rs, rbi)
    ret = _head(comb, ew1, eb1, elw, elb, ew2, eb2, ew3, eb3, es, ebi)
    col = lax.broadcasted_iota(jnp.int32, (comb.shape[0], 8), 1)
    out_ref[...] = jnp.where(col == 0, rtt, jnp.where(col == 1, ret, 0.0))


def _stage_d(agg, rin, gin, inv, d, nw, nb, head_ws):
    n = rin.shape[0]

    def body(agg_ref, rin_ref, gin_ref, inv_ref, d_ref, nw_ref, nb_ref, *hw):
        _stage_d_body(agg_ref, rin_ref, gin_ref, inv_ref, d_ref, nw_ref,
                      nb_ref, hw[:-1], hw[-1])

    return pl.pallas_call(
        body, out_shape=jax.ShapeDtypeStruct((n, 8), jnp.float32),
    )(agg, rin, gin, inv, d, nw, nb, *head_ws)


@functools.lru_cache(maxsize=None)
def _make_segsum(e_total, n_pad, w):
    epw = e_total // _NW
    nch = epw // _CH
    rps = n_pad // _NS
    mesh = plsc.VectorSubcoreMesh(core_axis_name="c", subcore_axis_name="s")

    @functools.partial(
        pl.kernel,
        out_type=jax.ShapeDtypeStruct((_NC, n_pad, w), jnp.float32),
        mesh=mesh,
        scratch_types=[
            pltpu.VMEM((nch, _CH), jnp.int32),
            pltpu.VMEM((nch, _CH), jnp.int32),
            pltpu.VMEM((2, _CH, w), jnp.float32),
            pltpu.VMEM_SHARED((n_pad, w), jnp.float32),
            pltpu.SemaphoreType.DMA,
            pltpu.SemaphoreType.DMA,
        ],
        compiler_params=pltpu.CompilerParams(use_tc_tiling_on_sc=False),
    )
    def segsum(p_hbm, src_hbm, dst_hbm, z_hbm, out_hbm,
               src_v, dst_v, rows_v, agg_sh, semg, sems):
        cid = lax.axis_index("c")
        sid = lax.axis_index("s")
        wid = sid * _NC + cid
        pltpu.sync_copy(z_hbm.at[pl.ds(sid * rps, rps)],
                        agg_sh.at[pl.ds(sid * rps, rps)])
        pltpu.sync_copy(src_hbm.at[wid], src_v)
        pltpu.sync_copy(dst_hbm.at[wid], dst_v)
        plsc.subcore_barrier()

        def g_start(i, b):
            pltpu.async_copy(p_hbm.at[src_v.at[i]], rows_v.at[b], semg)

        def g_wait(i, b):
            pltpu.make_async_copy(p_hbm.at[src_v.at[i]], rows_v.at[b],
                                  semg).wait()

        def s_start(i, b):
            pltpu.async_copy(rows_v.at[b], agg_sh.at[dst_v.at[i]], sems,
                             add=True)

        def s_wait(i, b):
            pltpu.make_async_copy(rows_v.at[b], agg_sh.at[dst_v.at[i]],
                                  sems).wait()

        g_start(0, 0)

        def body(i, carry):
            b = lax.rem(i, 2)
            g_wait(i, b)
            s_start(i, b)

            @pl.when(i > 0)
            def _():
                s_wait(i - 1, 1 - b)

            g_start(i + 1, 1 - b)
            return carry

        lax.fori_loop(0, nch - 1, body, 0)
        bl = (nch - 1) % 2
        g_wait(nch - 1, bl)
        s_start(nch - 1, bl)
        s_wait(nch - 2, 1 - bl)
        s_wait(nch - 1, bl)
        plsc.subcore_barrier()
        pltpu.sync_copy(agg_sh.at[pl.ds(sid * rps, rps)],
                        out_hbm.at[cid].at[pl.ds(sid * rps, rps)])

    return segsum


def kernel(x, edge_index, params):
    p = params
    n = x.shape[0]
    e_total = edge_index.shape[1]
    src3 = edge_index[0].reshape(_NW, e_total // _NW // _CH, _CH)
    dst3 = edge_index[1].reshape(_NW, e_total // _NW // _CH, _CH)

    row = lambda v: v.reshape(1, -1)
    d, p0, r0 = _stage_a(
        x, row(p['bn_w']), row(p['bn_b']), row(p['fw']),
        p['ft_w1'].T, row(p['ft_b1']), row(p['ft_ln1_w']), row(p['ft_ln1_b']),
        p['ft_w2'].T, row(p['ft_b2']), row(p['ft_ln2_w']), row(p['ft_ln2_b']),
        p['conv0_wl'].T, p['conv0_wr'].T)

    n_pad = ((n + 8 * _NS - 1) // (8 * _NS)) * (8 * _NS)
    z80 = jnp.zeros((n_pad, 80), jnp.float32)
    z64 = jnp.zeros((n_pad, 64), jnp.float32)
    segsum80 = _make_segsum(e_total, n_pad, 80)
    segsum64 = _make_segsum(e_total, n_pad, 64)

    agg0 = segsum80(p0, src3, dst3, z80)
    g1, p1, r1, inv = _stage_b(agg0, r0, row(p['norm0_w']), row(p['norm0_b']),
                               p['conv1_wl'].T, p['conv1_wr'].T)
    agg1 = segsum64(p1, src3, dst3, z64)
    g2, p2, r2 = _stage_c(agg1, r1, g1, inv, row(p['norm1_w']),
                          row(p['norm1_b']), p['conv2_wl'].T, p['conv2_wr'].T)
    agg2 = segsum64(p2, src3, dst3, z64)

    head_ws = []
    for pre in ['rtt', 'ret']:
        head_ws += [p[pre + '_w1'].T, row(p[pre + '_b1']),
                    row(p[pre + '_ln_w']), row(p[pre + '_ln_b']),
                    p[pre + '_w2'].T, row(p[pre + '_b2']),
                    p[pre + '_w3'].T, row(p[pre + '_b3']),
                    p[pre + '_scale'].reshape(1, 1), p[pre + '_bias'].reshape(1, 1)]
    o8 = _stage_d(agg2, r2, g2, inv, d, row(p['norm2_w']), row(p['norm2_b']),
                  head_ws)
    return o8[:, :2]


# speedup vs baseline: 11.5549x; 1.3220x over previous
import functools

import jax
import jax.numpy as jnp
from jax import lax
from jax.experimental import pallas as pl
from jax.experimental.pallas import tpu as pltpu
from jax.experimental.pallas import tpu_sc as plsc

_NC = 2
_NS = 16
_NW = _NC * _NS
_CH = 200


def _ln(h, w, b):
    mu = jnp.mean(h, axis=-1, keepdims=True)
    d = h - mu
    var = jnp.mean(d * d, axis=-1, keepdims=True)
    return d / jnp.sqrt(var + 1e-5) * w + b


def _l2n(h):
    nrm = jnp.sqrt(jnp.sum(h * h, axis=-1, keepdims=True))
    return h / jnp.maximum(nrm, 1e-12)


def _stage_a_body(x_ref, bnw_ref, bnb_ref, fw_ref, w1_ref, b1_ref, l1w_ref,
                  l1b_ref, w2_ref, b2_ref, l2w_ref, l2b_ref, wl0_ref, wr0_ref,
                  d_ref, p0_ref, r0_ref):
    x = x_ref[...]
    mu = jnp.mean(x, axis=0, keepdims=True)
    xc = x - mu
    var = jnp.mean(xc * xc, axis=0, keepdims=True)
    h = xc * (bnw_ref[...] / jnp.sqrt(var + 1e-5)) + bnb_ref[...]
    xw = h * (1.0 + 0.5 * jax.nn.sigmoid(fw_ref[...]))
    t = jnp.dot(xw, w1_ref[...], preferred_element_type=jnp.float32) + b1_ref[...]
    t = jnp.maximum(_ln(t, l1w_ref[...], l1b_ref[...]), 0.0)
    t = jnp.dot(t, w2_ref[...], preferred_element_type=jnp.float32) + b2_ref[...]
    d_ref[...] = jnp.maximum(_ln(t, l2w_ref[...], l2b_ref[...]), 0.0)
    p = jnp.dot(xw, wl0_ref[...], preferred_element_type=jnp.float32)
    col = lax.broadcasted_iota(jnp.int32, (x.shape[0], 16), 1)
    ones_col = jnp.where(col == 0, 1.0, 0.0)
    p0_ref[...] = jnp.concatenate([p, ones_col], axis=1)
    r0_ref[...] = jnp.dot(xw, wr0_ref[...], preferred_element_type=jnp.float32)


def _stage_a(x, bnw, bnb, fw, w1, b1, l1w, l1b, w2, b2, l2w, l2b, wl0, wr0):
    n = x.shape[0]
    return pl.pallas_call(
        _stage_a_body,
        out_shape=[
            jax.ShapeDtypeStruct((n, 64), jnp.float32),
            jax.ShapeDtypeStruct((n, 80), jnp.float32),
            jax.ShapeDtypeStruct((n, 64), jnp.float32),
        ],
    )(x, bnw, bnb, fw, w1, b1, l1w, l1b, w2, b2, l2w, l2b, wl0, wr0)


def _stage_b_body(agg_ref, r0_ref, nw_ref, nb_ref, wl_ref, wr_ref,
                  g_ref, p_ref, r_ref, inv_ref):
    n = r0_ref.shape[0]
    agg = agg_ref[0, :n] + agg_ref[1, :n]
    deg = jnp.sum(agg[:, 64:80], axis=-1, keepdims=True)
    inv = 1.0 / jnp.maximum(deg, 1.0)
    sage = agg[:, :64] * inv + r0_ref[...]
    g = jnp.maximum(_ln(_l2n(sage), nw_ref[...], nb_ref[...]), 0.0)
    g_ref[...] = g
    p_ref[...] = jnp.dot(g, wl_ref[...], preferred_element_type=jnp.float32)
    r_ref[...] = jnp.dot(g, wr_ref[...], preferred_element_type=jnp.float32)
    inv_ref[...] = jnp.broadcast_to(inv, (agg.shape[0], 64))


def _stage_b(agg0, r0, nw, nb, wl, wr):
    n = r0.shape[0]
    s64 = jax.ShapeDtypeStruct((n, 64), jnp.float32)
    return pl.pallas_call(
        _stage_b_body, out_shape=[s64, s64, s64, s64],
    )(agg0, r0, nw, nb, wl, wr)


def _stage_c_body(agg_ref, rin_ref, gin_ref, inv_ref, nw_ref, nb_ref,
                  wl_ref, wr_ref, g_ref, p_ref, r_ref):
    n = rin_ref.shape[0]
    agg = agg_ref[0, :n] + agg_ref[1, :n]
    sage = agg * inv_ref[...] + rin_ref[...]
    g = jnp.maximum(_ln(_l2n(sage), nw_ref[...], nb_ref[...]), 0.0)
    g = g + 0.1 * gin_ref[...]
    g_ref[...] = g
    p_ref[...] = jnp.dot(g, wl_ref[...], preferred_element_type=jnp.float32)
    r_ref[...] = jnp.dot(g, wr_ref[...], preferred_element_type=jnp.float32)


def _stage_c(agg, rin, gin, inv, nw, nb, wl, wr):
    n = rin.shape[0]
    s64 = jax.ShapeDtypeStruct((n, 64), jnp.float32)
    return pl.pallas_call(
        _stage_c_body, out_shape=[s64, s64, s64],
    )(agg, rin, gin, inv, nw, nb, wl, wr)


def _head(comb, w1, b1, lw, lb, w2, b2, w3, b3, scale, bias):
    o = jnp.dot(comb, w1, preferred_element_type=jnp.float32) + b1
    o = jnp.maximum(_ln(o, lw, lb), 0.0)
    o = jnp.maximum(jnp.dot(o, w2, preferred_element_type=jnp.float32) + b2, 0.0)
    o = jnp.dot(o, w3, preferred_element_type=jnp.float32) + b3
    return o * jnp.abs(scale) + bias


def _stage_d_body(agg_ref, rin_ref, gin_ref, inv_ref, d_ref, nw_ref, nb_ref,
                  hw_refs, out_ref):
    n = rin_ref.shape[0]
    agg = agg_ref[0, :n] + agg_ref[1, :n]
    sage = agg * inv_ref[...] + rin_ref[...]
    g = jnp.maximum(_ln(_l2n(sage), nw_ref[...], nb_ref[...]), 0.0)
    g = g + 0.1 * gin_ref[...]
    comb = jnp.concatenate([d_ref[...], g], axis=1)
    (rw1, rb1, rlw, rlb, rw2, rb2, rw3, rb3, rs, rbi,
     ew1, eb1, elw, elb, ew2, eb2, ew3, eb3, es, ebi) = [h[...] for h in hw_refs]
    rtt = _head(comb, rw1, rb1, rlw, rlb, rw2, rb2, rw3, rb3, rs, rbi)
    ret = _head(comb, ew1, eb1, elw, elb, ew2, eb2, ew3, eb3, es, ebi)
    col = lax.broadcasted_iota(jnp.int32, (comb.shape[0], 8), 1)
    out_ref[...] = jnp.where(col == 0, rtt, jnp.where(col == 1, ret, 0.0))


def _stage_d(agg, rin, gin, inv, d, nw, nb, head_ws):
    n = rin.shape[0]

    def body(agg_ref, rin_ref, gin_ref, inv_ref, d_ref, nw_ref, nb_ref, *hw):
        _stage_d_body(agg_ref, rin_ref, gin_ref, inv_ref, d_ref, nw_ref,
                      nb_ref, hw[:-1], hw[-1])

    return pl.pallas_call(
        body, out_shape=jax.ShapeDtypeStruct((n, 8), jnp.float32),
    )(agg, rin, gin, inv, d, nw, nb, *head_ws)


@functools.lru_cache(maxsize=None)
def _make_segsum(e_total, n_pad, w):
    epw = e_total // _NW
    nch = epw // _CH
    rps = n_pad // _NS
    mesh = plsc.VectorSubcoreMesh(core_axis_name="c", subcore_axis_name="s")

    @functools.partial(
        pl.kernel,
        out_type=jax.ShapeDtypeStruct((_NC, n_pad, w), jnp.float32),
        mesh=mesh,
        scratch_types=[
            pltpu.VMEM((nch, _CH), jnp.int32),
            pltpu.VMEM((nch, _CH), jnp.int32),
            pltpu.VMEM((2, _CH, w), jnp.float32),
            pltpu.VMEM_SHARED((n_pad, w), jnp.float32),
            pltpu.SemaphoreType.DMA,
            pltpu.SemaphoreType.DMA,
        ],
        compiler_params=pltpu.CompilerParams(use_tc_tiling_on_sc=False),
    )
    def segsum(p_hbm, src_hbm, dst_hbm, z_hbm, out_hbm,
               src_v, dst_v, rows_v, agg_sh, semg, sems):
        cid = lax.axis_index("c")
        sid = lax.axis_index("s")
        wid = sid * _NC + cid
        pltpu.sync_copy(z_hbm.at[pl.ds(sid * rps, rps)],
                        agg_sh.at[pl.ds(sid * rps, rps)])
        pltpu.sync_copy(src_hbm.at[wid], src_v)
        pltpu.sync_copy(dst_hbm.at[wid], dst_v)
        plsc.subcore_barrier()

        def g_start(i, b):
            pltpu.async_copy(p_hbm.at[src_v.at[i]], rows_v.at[b], semg)

        def g_wait(i, b):
            pltpu.make_async_copy(p_hbm.at[src_v.at[i]], rows_v.at[b],
                                  semg).wait()

        def s_start(i, b):
            pltpu.async_copy(rows_v.at[b], agg_sh.at[dst_v.at[i]], sems,
                             add=True)

        def s_wait(i, b):
            pltpu.make_async_copy(rows_v.at[b], agg_sh.at[dst_v.at[i]],
                                  sems).wait()

        g_start(0, 0)

        def body(i, carry):
            b = lax.rem(i, 2)
            g_wait(i, b)
            s_start(i, b)

            @pl.when(i > 0)
            def _():
                s_wait(i - 1, 1 - b)

            g_start(i + 1, 1 - b)
            return carry

        lax.fori_loop(0, nch - 1, body, 0)
        bl = (nch - 1) % 2
        g_wait(nch - 1, bl)
        s_start(nch - 1, bl)
        s_wait(nch - 2, 1 - bl)
        s_wait(nch - 1, bl)
        plsc.subcore_barrier()
        pltpu.sync_copy(agg_sh.at[pl.ds(sid * rps, rps)],
                        out_hbm.at[cid].at[pl.ds(sid * rps, rps)])

    return segsum


def kernel(x, edge_index, params):
    p = params
    n = x.shape[0]
    e_total = edge_index.shape[1]
    src3 = edge_index[0].reshape(_NW, e_total // _NW // _CH, _CH)
    dst3 = edge_index[1].reshape(_NW, e_total // _NW // _CH, _CH)

    row = lambda v: v.reshape(1, -1)
    d, p0, r0 = _stage_a(
        x, row(p['bn_w']), row(p['bn_b']), row(p['fw']),
        p['ft_w1'].T, row(p['ft_b1']), row(p['ft_ln1_w']), row(p['ft_ln1_b']),
        p['ft_w2'].T, row(p['ft_b2']), row(p['ft_ln2_w']), row(p['ft_ln2_b']),
        p['conv0_wl'].T, p['conv0_wr'].T)

    n_pad = ((n + 8 * _NS - 1) // (8 * _NS)) * (8 * _NS)
    z80 = jnp.zeros((n_pad, 80), jnp.float32)
    z64 = jnp.zeros((n_pad, 64), jnp.float32)
    segsum80 = _make_segsum(e_total, n_pad, 80)
    segsum64 = _make_segsum(e_total, n_pad, 64)

    agg0 = segsum80(p0, src3, dst3, z80)
    g1, p1, r1, inv = _stage_b(agg0, r0, row(p['norm0_w']), row(p['norm0_b']),
                               p['conv1_wl'].T, p['conv1_wr'].T)
    agg1 = segsum64(p1, src3, dst3, z64)
    g2, p2, r2 = _stage_c(agg1, r1, g1, inv, row(p['norm1_w']),
                          row(p['norm1_b']), p['conv2_wl'].T, p['conv2_wr'].T)
    agg2 = segsum64(p2, src3, dst3, z64)

    head_ws = []
    for pre in ['rtt', 'ret']:
        head_ws += [p[pre + '_w1'].T, row(p[pre + '_b1']),
                    row(p[pre + '_ln_w']), row(p[pre + '_ln_b']),
                    p[pre + '_w2'].T, row(p[pre + '_b2']),
                    p[pre + '_w3'].T, row(p[pre + '_b3']),
                    p[pre + '_scale'].reshape(1, 1), p[pre + '_bias'].reshape(1, 1)]
    o8 = _stage_d(agg2, r2, g2, inv, d, row(p['norm2_w']), row(p['norm2_b']),
                  head_ws)
    return o8[:, :2]


# speedup vs baseline: 12.1288x; 1.0497x over previous
import functools

import jax
import jax.numpy as jnp
from jax import lax
from jax.experimental import pallas as pl
from jax.experimental.pallas import tpu as pltpu
from jax.experimental.pallas import tpu_sc as plsc

_NC = 2
_NS = 16
_NW = _NC * _NS
_CH = 200


def _ln(h, w, b):
    mu = jnp.mean(h, axis=-1, keepdims=True)
    d = h - mu
    var = jnp.mean(d * d, axis=-1, keepdims=True)
    return d / jnp.sqrt(var + 1e-5) * w + b


def _l2n(h):
    nrm = jnp.sqrt(jnp.sum(h * h, axis=-1, keepdims=True))
    return h / jnp.maximum(nrm, 1e-12)


def _stage_a_body(x_ref, bnw_ref, bnb_ref, fw_ref, w1_ref, b1_ref, l1w_ref,
                  l1b_ref, w2_ref, b2_ref, l2w_ref, l2b_ref, wl0_ref, wr0_ref,
                  d_ref, p0_ref, r0_ref):
    x = x_ref[...]
    mu = jnp.mean(x, axis=0, keepdims=True)
    xc = x - mu
    var = jnp.mean(xc * xc, axis=0, keepdims=True)
    h = xc * (bnw_ref[...] / jnp.sqrt(var + 1e-5)) + bnb_ref[...]
    xw = h * (1.0 + 0.5 * jax.nn.sigmoid(fw_ref[...]))
    t = jnp.dot(xw, w1_ref[...], preferred_element_type=jnp.float32) + b1_ref[...]
    t = jnp.maximum(_ln(t, l1w_ref[...], l1b_ref[...]), 0.0)
    t = jnp.dot(t, w2_ref[...], preferred_element_type=jnp.float32) + b2_ref[...]
    d_ref[...] = jnp.maximum(_ln(t, l2w_ref[...], l2b_ref[...]), 0.0)
    p = jnp.dot(xw, wl0_ref[...], preferred_element_type=jnp.float32)
    col = lax.broadcasted_iota(jnp.int32, (x.shape[0], 16), 1)
    ones_col = jnp.where(col == 0, 1.0, 0.0)
    p0_ref[...] = jnp.concatenate([p, ones_col], axis=1)
    r0_ref[...] = jnp.dot(xw, wr0_ref[...], preferred_element_type=jnp.float32)


def _stage_a(x, bnw, bnb, fw, w1, b1, l1w, l1b, w2, b2, l2w, l2b, wl0, wr0):
    n = x.shape[0]
    return pl.pallas_call(
        _stage_a_body,
        out_shape=[
            jax.ShapeDtypeStruct((n, 64), jnp.float32),
            jax.ShapeDtypeStruct((n, 80), jnp.float32),
            jax.ShapeDtypeStruct((n, 64), jnp.float32),
        ],
    )(x, bnw, bnb, fw, w1, b1, l1w, l1b, w2, b2, l2w, l2b, wl0, wr0)


def _stage_b_body(agg_ref, r0_ref, nw_ref, nb_ref, wl_ref, wr_ref,
                  g_ref, p_ref, r_ref, inv_ref):
    n = r0_ref.shape[0]
    agg = agg_ref[0, :n] + agg_ref[1, :n]
    deg = jnp.sum(agg[:, 64:80], axis=-1, keepdims=True)
    inv = 1.0 / jnp.maximum(deg, 1.0)
    sage = agg[:, :64] * inv + r0_ref[...]
    g = jnp.maximum(_ln(_l2n(sage), nw_ref[...], nb_ref[...]), 0.0)
    g_ref[...] = g
    p_ref[...] = jnp.dot(g, wl_ref[...], preferred_element_type=jnp.float32)
    r_ref[...] = jnp.dot(g, wr_ref[...], preferred_element_type=jnp.float32)
    inv_ref[...] = jnp.broadcast_to(inv, (agg.shape[0], 64))


def _stage_b(agg0, r0, nw, nb, wl, wr):
    n = r0.shape[0]
    s64 = jax.ShapeDtypeStruct((n, 64), jnp.float32)
    return pl.pallas_call(
        _stage_b_body, out_shape=[s64, s64, s64, s64],
    )(agg0, r0, nw, nb, wl, wr)


def _stage_c_body(agg_ref, rin_ref, gin_ref, inv_ref, nw_ref, nb_ref,
                  wl_ref, wr_ref, g_ref, p_ref, r_ref):
    n = rin_ref.shape[0]
    agg = agg_ref[0, :n] + agg_ref[1, :n]
    sage = agg * inv_ref[...] + rin_ref[...]
    g = jnp.maximum(_ln(_l2n(sage), nw_ref[...], nb_ref[...]), 0.0)
    g = g + 0.1 * gin_ref[...]
    g_ref[...] = g
    p_ref[...] = jnp.dot(g, wl_ref[...], preferred_element_type=jnp.float32)
    r_ref[...] = jnp.dot(g, wr_ref[...], preferred_element_type=jnp.float32)


def _stage_c(agg, rin, gin, inv, nw, nb, wl, wr):
    n = rin.shape[0]
    s64 = jax.ShapeDtypeStruct((n, 64), jnp.float32)
    return pl.pallas_call(
        _stage_c_body, out_shape=[s64, s64, s64],
    )(agg, rin, gin, inv, nw, nb, wl, wr)


def _head(comb, w1, b1, lw, lb, w2, b2, w3, b3, scale, bias):
    o = jnp.dot(comb, w1, preferred_element_type=jnp.float32) + b1
    o = jnp.maximum(_ln(o, lw, lb), 0.0)
    o = jnp.maximum(jnp.dot(o, w2, preferred_element_type=jnp.float32) + b2, 0.0)
    o = jnp.dot(o, w3, preferred_element_type=jnp.float32) + b3
    return o * jnp.abs(scale) + bias


def _stage_d_body(agg_ref, rin_ref, gin_ref, inv_ref, d_ref, nw_ref, nb_ref,
                  hw_refs, out_ref):
    n = rin_ref.shape[0]
    agg = agg_ref[0, :n] + agg_ref[1, :n]
    sage = agg * inv_ref[...] + rin_ref[...]
    g = jnp.maximum(_ln(_l2n(sage), nw_ref[...], nb_ref[...]), 0.0)
    g = g + 0.1 * gin_ref[...]
    comb = jnp.concatenate([d_ref[...], g], axis=1)
    (rw1, rb1, rlw, rlb, rw2, rb2, rw3, rb3, rs, rbi,
     ew1, eb1, elw, elb, ew2, eb2, ew3, eb3, es, ebi) = [h[...] for h in hw_refs]
    rtt = _head(comb, rw1, rb1, rlw, rlb, rw2, rb2, rw3, rb3, rs, rbi)
    ret = _head(comb, ew1, eb1, elw, elb, ew2, eb2, ew3, eb3, es, ebi)
    col = lax.broadcasted_iota(jnp.int32, (comb.shape[0], 8), 1)
    out_ref[...] = jnp.where(col == 0, rtt, jnp.where(col == 1, ret, 0.0))


def _stage_d(agg, rin, gin, inv, d, nw, nb, head_ws):
    n = rin.shape[0]

    def body(agg_ref, rin_ref, gin_ref, inv_ref, d_ref, nw_ref, nb_ref, *hw):
        _stage_d_body(agg_ref, rin_ref, gin_ref, inv_ref, d_ref, nw_ref,
                      nb_ref, hw[:-1], hw[-1])

    return pl.pallas_call(
        body, out_shape=jax.ShapeDtypeStruct((n, 8), jnp.float32),
    )(agg, rin, gin, inv, d, nw, nb, *head_ws)


@functools.lru_cache(maxsize=None)
def _make_segsum(e_total, n_pad, w, ch):
    epw = e_total // _NW
    nch = epw // ch
    rps = n_pad // _NS
    mesh = plsc.VectorSubcoreMesh(core_axis_name="c", subcore_axis_name="s")

    @functools.partial(
        pl.kernel,
        out_type=jax.ShapeDtypeStruct((_NC, n_pad, w), jnp.float32),
        mesh=mesh,
        scratch_types=[
            pltpu.VMEM((nch, ch), jnp.int32),
            pltpu.VMEM((nch, ch), jnp.int32),
            pltpu.VMEM((2, ch, w), jnp.float32),
            pltpu.VMEM_SHARED((n_pad, w), jnp.float32),
            pltpu.SemaphoreType.DMA,
            pltpu.SemaphoreType.DMA,
        ],
        compiler_params=pltpu.CompilerParams(use_tc_tiling_on_sc=False),
    )
    def segsum(p_hbm, src_hbm, dst_hbm, z_hbm, out_hbm,
               src_v, dst_v, rows_v, agg_sh, semg, sems):
        cid = lax.axis_index("c")
        sid = lax.axis_index("s")
        wid = sid * _NC + cid
        pltpu.sync_copy(z_hbm.at[pl.ds(sid * rps, rps)],
                        agg_sh.at[pl.ds(sid * rps, rps)])
        pltpu.sync_copy(src_hbm.at[wid], src_v)
        pltpu.sync_copy(dst_hbm.at[wid], dst_v)
        plsc.subcore_barrier()

        def g_start(i, b):
            pltpu.async_copy(p_hbm.at[src_v.at[i]], rows_v.at[b], semg)

        def g_wait(i, b):
            pltpu.make_async_copy(p_hbm.at[src_v.at[i]], rows_v.at[b],
                                  semg).wait()

        def s_start(i, b):
            pltpu.async_copy(rows_v.at[b], agg_sh.at[dst_v.at[i]], sems,
                             add=True)

        def s_wait(i, b):
            pltpu.make_async_copy(rows_v.at[b], agg_sh.at[dst_v.at[i]],
                                  sems).wait()

        g_start(0, 0)

        def body(i, carry):
            b = lax.rem(i, 2)
            g_wait(i, b)
            s_start(i, b)

            @pl.when(i > 0)
            def _():
                s_wait(i - 1, 1 - b)

            g_start(i + 1, 1 - b)
            return carry

        lax.fori_loop(0, nch - 1, body, 0)
        bl = (nch - 1) % 2
        g_wait(nch - 1, bl)
        s_start(nch - 1, bl)
        s_wait(nch - 2, 1 - bl)
        s_wait(nch - 1, bl)
        plsc.subcore_barrier()
        pltpu.sync_copy(agg_sh.at[pl.ds(sid * rps, rps)],
                        out_hbm.at[cid].at[pl.ds(sid * rps, rps)])

    return segsum


def kernel(x, edge_index, params):
    p = params
    n = x.shape[0]
    e_total = edge_index.shape[1]
    ch80, ch64 = 200, 400
    src80 = edge_index[0].reshape(_NW, e_total // _NW // ch80, ch80)
    dst80 = edge_index[1].reshape(_NW, e_total // _NW // ch80, ch80)
    src64 = edge_index[0].reshape(_NW, e_total // _NW // ch64, ch64)
    dst64 = edge_index[1].reshape(_NW, e_total // _NW // ch64, ch64)

    row = lambda v: v.reshape(1, -1)
    d, p0, r0 = _stage_a(
        x, row(p['bn_w']), row(p['bn_b']), row(p['fw']),
        p['ft_w1'].T, row(p['ft_b1']), row(p['ft_ln1_w']), row(p['ft_ln1_b']),
        p['ft_w2'].T, row(p['ft_b2']), row(p['ft_ln2_w']), row(p['ft_ln2_b']),
        p['conv0_wl'].T, p['conv0_wr'].T)

    n_pad = ((n + 8 * _NS - 1) // (8 * _NS)) * (8 * _NS)
    z80 = jnp.zeros((n_pad, 80), jnp.float32)
    z64 = jnp.zeros((n_pad, 64), jnp.float32)
    segsum80 = _make_segsum(e_total, n_pad, 80, ch80)
    segsum64 = _make_segsum(e_total, n_pad, 64, ch64)

    agg0 = segsum80(p0, src80, dst80, z80)
    g1, p1, r1, inv = _stage_b(agg0, r0, row(p['norm0_w']), row(p['norm0_b']),
                               p['conv1_wl'].T, p['conv1_wr'].T)
    agg1 = segsum64(p1, src64, dst64, z64)
    g2, p2, r2 = _stage_c(agg1, r1, g1, inv, row(p['norm1_w']),
                          row(p['norm1_b']), p['conv2_wl'].T, p['conv2_wr'].T)
    agg2 = segsum64(p2, src64, dst64, z64)

    head_ws = []
    for pre in ['rtt', 'ret']:
        head_ws += [p[pre + '_w1'].T, row(p[pre + '_b1']),
                    row(p[pre + '_ln_w']), row(p[pre + '_ln_b']),
                    p[pre + '_w2'].T, row(p[pre + '_b2']),
                    p[pre + '_w3'].T, row(p[pre + '_b3']),
                    p[pre + '_scale'].reshape(1, 1), p[pre + '_bias'].reshape(1, 1)]
    o8 = _stage_d(agg2, r2, g2, inv, d, row(p['norm2_w']), row(p['norm2_b']),
                  head_ws)
    return o8[:, :2]


# speedup vs baseline: 12.4121x; 1.0234x over previous
import functools

import jax
import jax.numpy as jnp
from jax import lax
from jax.experimental import pallas as pl
from jax.experimental.pallas import tpu as pltpu
from jax.experimental.pallas import tpu_sc as plsc

_NC = 2
_NS = 16
_NW = _NC * _NS
_CH = 200


def _ln(h, w, b):
    mu = jnp.mean(h, axis=-1, keepdims=True)
    d = h - mu
    var = jnp.mean(d * d, axis=-1, keepdims=True)
    return d / jnp.sqrt(var + 1e-5) * w + b


def _l2n(h):
    nrm = jnp.sqrt(jnp.sum(h * h, axis=-1, keepdims=True))
    return h / jnp.maximum(nrm, 1e-12)


def _stage_a_body(x_ref, bnw_ref, bnb_ref, fw_ref, w1_ref, b1_ref, l1w_ref,
                  l1b_ref, w2_ref, b2_ref, l2w_ref, l2b_ref, wl0_ref, wr0_ref,
                  d_ref, p0_ref, r0_ref):
    x = x_ref[...]
    mu = jnp.mean(x, axis=0, keepdims=True)
    xc = x - mu
    var = jnp.mean(xc * xc, axis=0, keepdims=True)
    h = xc * (bnw_ref[...] / jnp.sqrt(var + 1e-5)) + bnb_ref[...]
    xw = h * (1.0 + 0.5 * jax.nn.sigmoid(fw_ref[...]))
    t = jnp.dot(xw, w1_ref[...], preferred_element_type=jnp.float32) + b1_ref[...]
    t = jnp.maximum(_ln(t, l1w_ref[...], l1b_ref[...]), 0.0)
    t = jnp.dot(t, w2_ref[...], preferred_element_type=jnp.float32) + b2_ref[...]
    d_ref[...] = jnp.maximum(_ln(t, l2w_ref[...], l2b_ref[...]), 0.0)
    p = jnp.dot(xw, wl0_ref[...], preferred_element_type=jnp.float32)
    col = lax.broadcasted_iota(jnp.int32, (x.shape[0], 16), 1)
    ones_col = jnp.where(col == 0, 1.0, 0.0)
    p0_ref[...] = jnp.concatenate([p, ones_col], axis=1)
    r0_ref[...] = jnp.dot(xw, wr0_ref[...], preferred_element_type=jnp.float32)


def _stage_a(x, bnw, bnb, fw, w1, b1, l1w, l1b, w2, b2, l2w, l2b, wl0, wr0):
    n = x.shape[0]
    return pl.pallas_call(
        _stage_a_body,
        out_shape=[
            jax.ShapeDtypeStruct((n, 64), jnp.float32),
            jax.ShapeDtypeStruct((n, 80), jnp.float32),
            jax.ShapeDtypeStruct((n, 64), jnp.float32),
        ],
    )(x, bnw, bnb, fw, w1, b1, l1w, l1b, w2, b2, l2w, l2b, wl0, wr0)


def _stage_b_body(agg_ref, r0_ref, nw_ref, nb_ref, wl_ref, wr_ref,
                  g_ref, p_ref, r_ref, inv_ref):
    n = r0_ref.shape[0]
    agg = agg_ref[0, :n] + agg_ref[1, :n]
    deg = jnp.sum(agg[:, 64:80], axis=-1, keepdims=True)
    inv = 1.0 / jnp.maximum(deg, 1.0)
    sage = agg[:, :64] * inv + r0_ref[...]
    g = jnp.maximum(_ln(_l2n(sage), nw_ref[...], nb_ref[...]), 0.0)
    g_ref[...] = g
    p_ref[...] = jnp.dot(g, wl_ref[...], preferred_element_type=jnp.float32)
    r_ref[...] = jnp.dot(g, wr_ref[...], preferred_element_type=jnp.float32)
    inv_ref[...] = jnp.broadcast_to(inv, (agg.shape[0], 64))


def _stage_b(agg0, r0, nw, nb, wl, wr):
    n = r0.shape[0]
    s64 = jax.ShapeDtypeStruct((n, 64), jnp.float32)
    return pl.pallas_call(
        _stage_b_body, out_shape=[s64, s64, s64, s64],
    )(agg0, r0, nw, nb, wl, wr)


def _stage_c_body(agg_ref, rin_ref, gin_ref, inv_ref, nw_ref, nb_ref,
                  wl_ref, wr_ref, g_ref, p_ref, r_ref):
    n = rin_ref.shape[0]
    agg = agg_ref[0, :n] + agg_ref[1, :n]
    sage = agg * inv_ref[...] + rin_ref[...]
    g = jnp.maximum(_ln(_l2n(sage), nw_ref[...], nb_ref[...]), 0.0)
    g = g + 0.1 * gin_ref[...]
    g_ref[...] = g
    p_ref[...] = jnp.dot(g, wl_ref[...], preferred_element_type=jnp.float32)
    r_ref[...] = jnp.dot(g, wr_ref[...], preferred_element_type=jnp.float32)


def _stage_c(agg, rin, gin, inv, nw, nb, wl, wr):
    n = rin.shape[0]
    s64 = jax.ShapeDtypeStruct((n, 64), jnp.float32)
    return pl.pallas_call(
        _stage_c_body, out_shape=[s64, s64, s64],
    )(agg, rin, gin, inv, nw, nb, wl, wr)


def _head(comb, w1, b1, lw, lb, w2, b2, w3, b3, scale, bias):
    o = jnp.dot(comb, w1, preferred_element_type=jnp.float32) + b1
    o = jnp.maximum(_ln(o, lw, lb), 0.0)
    o = jnp.maximum(jnp.dot(o, w2, preferred_element_type=jnp.float32) + b2, 0.0)
    o = jnp.dot(o, w3, preferred_element_type=jnp.float32) + b3
    return o * jnp.abs(scale) + bias


def _stage_d_body(agg_ref, rin_ref, gin_ref, inv_ref, d_ref, nw_ref, nb_ref,
                  hw_refs, out_ref):
    n = rin_ref.shape[0]
    agg = agg_ref[0, :n] + agg_ref[1, :n]
    sage = agg * inv_ref[...] + rin_ref[...]
    g = jnp.maximum(_ln(_l2n(sage), nw_ref[...], nb_ref[...]), 0.0)
    g = g + 0.1 * gin_ref[...]
    comb = jnp.concatenate([d_ref[...], g], axis=1)
    (rw1, rb1, rlw, rlb, rw2, rb2, rw3, rb3, rs, rbi,
     ew1, eb1, elw, elb, ew2, eb2, ew3, eb3, es, ebi) = [h[...] for h in hw_refs]
    rtt = _head(comb, rw1, rb1, rlw, rlb, rw2, rb2, rw3, rb3, rs, rbi)
    ret = _head(comb, ew1, eb1, elw, elb, ew2, eb2, ew3, eb3, es, ebi)
    col = lax.broadcasted_iota(jnp.int32, (comb.shape[0], 8), 1)
    out_ref[...] = jnp.where(col == 0, rtt, jnp.where(col == 1, ret, 0.0))


def _stage_d(agg, rin, gin, inv, d, nw, nb, head_ws):
    n = rin.shape[0]

    def body(agg_ref, rin_ref, gin_ref, inv_ref, d_ref, nw_ref, nb_ref, *hw):
        _stage_d_body(agg_ref, rin_ref, gin_ref, inv_ref, d_ref, nw_ref,
                      nb_ref, hw[:-1], hw[-1])

    return pl.pallas_call(
        body, out_shape=jax.ShapeDtypeStruct((n, 8), jnp.float32),
    )(agg, rin, gin, inv, d, nw, nb, *head_ws)


@functools.lru_cache(maxsize=None)
def _make_segsum(e_total, n_pad, w, ch):
    epw = e_total // _NW
    nch = epw // ch
    rps = n_pad // _NS
    mesh = plsc.VectorSubcoreMesh(core_axis_name="c", subcore_axis_name="s")

    @functools.partial(
        pl.kernel,
        out_type=jax.ShapeDtypeStruct((_NC, n_pad, w), jnp.float32),
        mesh=mesh,
        scratch_types=[
            pltpu.VMEM((nch, ch), jnp.int32),
            pltpu.VMEM((nch, ch), jnp.int32),
            pltpu.VMEM((2, ch, w), jnp.float32),
            pltpu.VMEM_SHARED((n_pad, w), jnp.float32),
            pltpu.SemaphoreType.DMA,
            pltpu.SemaphoreType.DMA,
            pltpu.SemaphoreType.DMA,
        ],
        compiler_params=pltpu.CompilerParams(use_tc_tiling_on_sc=False),
    )
    def segsum(p_hbm, src_hbm, dst_hbm, z_hbm, out_hbm,
               src_v, dst_v, rows_v, agg_sh, semg, sems, semz):
        cid = lax.axis_index("c")
        sid = lax.axis_index("s")
        wid = sid * _NC + cid
        pltpu.async_copy(z_hbm.at[pl.ds(sid * rps, rps)],
                         agg_sh.at[pl.ds(sid * rps, rps)], semz)
        pltpu.sync_copy(src_hbm.at[wid], src_v)
        pltpu.sync_copy(dst_hbm.at[wid], dst_v)

        def g_start(i, b):
            pltpu.async_copy(p_hbm.at[src_v.at[i]], rows_v.at[b], semg)

        def g_wait(i, b):
            pltpu.make_async_copy(p_hbm.at[src_v.at[i]], rows_v.at[b],
                                  semg).wait()

        def s_start(i, b):
            pltpu.async_copy(rows_v.at[b], agg_sh.at[dst_v.at[i]], sems,
                             add=True)

        def s_wait(i, b):
            pltpu.make_async_copy(rows_v.at[b], agg_sh.at[dst_v.at[i]],
                                  sems).wait()

        g_start(0, 0)
        pltpu.make_async_copy(z_hbm.at[pl.ds(sid * rps, rps)],
                              agg_sh.at[pl.ds(sid * rps, rps)], semz).wait()
        plsc.subcore_barrier()

        def body(i, carry):
            b = lax.rem(i, 2)
            g_wait(i, b)
            s_start(i, b)

            @pl.when(i > 0)
            def _():
                s_wait(i - 1, 1 - b)

            g_start(i + 1, 1 - b)
            return carry

        lax.fori_loop(0, nch - 1, body, 0)
        bl = (nch - 1) % 2
        g_wait(nch - 1, bl)
        s_start(nch - 1, bl)
        s_wait(nch - 2, 1 - bl)
        s_wait(nch - 1, bl)
        plsc.subcore_barrier()
        pltpu.sync_copy(agg_sh.at[pl.ds(sid * rps, rps)],
                        out_hbm.at[cid].at[pl.ds(sid * rps, rps)])

    return segsum


def kernel(x, edge_index, params):
    p = params
    n = x.shape[0]
    e_total = edge_index.shape[1]
    ch80, ch64 = 200, 400
    src80 = edge_index[0].reshape(_NW, e_total // _NW // ch80, ch80)
    dst80 = edge_index[1].reshape(_NW, e_total // _NW // ch80, ch80)
    src64 = edge_index[0].reshape(_NW, e_total // _NW // ch64, ch64)
    dst64 = edge_index[1].reshape(_NW, e_total // _NW // ch64, ch64)

    row = lambda v: v.reshape(1, -1)
    d, p0, r0 = _stage_a(
        x, row(p['bn_w']), row(p['bn_b']), row(p['fw']),
        p['ft_w1'].T, row(p['ft_b1']), row(p['ft_ln1_w']), row(p['ft_ln1_b']),
        p['ft_w2'].T, row(p['ft_b2']), row(p['ft_ln2_w']), row(p['ft_ln2_b']),
        p['conv0_wl'].T, p['conv0_wr'].T)

    n_pad = ((n + 8 * _NS - 1) // (8 * _NS)) * (8 * _NS)
    z80 = jnp.zeros((n_pad, 80), jnp.float32)
    z64 = jnp.zeros((n_pad, 64), jnp.float32)
    segsum80 = _make_segsum(e_total, n_pad, 80, ch80)
    segsum64 = _make_segsum(e_total, n_pad, 64, ch64)

    agg0 = segsum80(p0, src80, dst80, z80)
    g1, p1, r1, inv = _stage_b(agg0, r0, row(p['norm0_w']), row(p['norm0_b']),
                               p['conv1_wl'].T, p['conv1_wr'].T)
    agg1 = segsum64(p1, src64, dst64, z64)
    g2, p2, r2 = _stage_c(agg1, r1, g1, inv, row(p['norm1_w']),
                          row(p['norm1_b']), p['conv2_wl'].T, p['conv2_wr'].T)
    agg2 = segsum64(p2, src64, dst64, z64)

    head_ws = []
    for pre in ['rtt', 'ret']:
        head_ws += [p[pre + '_w1'].T, row(p[pre + '_b1']),
                    row(p[pre + '_ln_w']), row(p[pre + '_ln_b']),
                    p[pre + '_w2'].T, row(p[pre + '_b2']),
                    p[pre + '_w3'].T, row(p[pre + '_b3']),
                    p[pre + '_scale'].reshape(1, 1), p[pre + '_bias'].reshape(1, 1)]
    o8 = _stage_d(agg2, r2, g2, inv, d, row(p['norm2_w']), row(p['norm2_b']),
                  head_ws)
    return o8[:, :2]


# speedup vs baseline: 12.4421x; 1.0024x over previous
import functools

import jax
import jax.numpy as jnp
from jax import lax
from jax.experimental import pallas as pl
from jax.experimental.pallas import tpu as pltpu
from jax.experimental.pallas import tpu_sc as plsc

_NC = 2
_NS = 16
_NW = _NC * _NS
_CH = 200


def _ln(h, w, b):
    mu = jnp.mean(h, axis=-1, keepdims=True)
    d = h - mu
    var = jnp.mean(d * d, axis=-1, keepdims=True)
    return d / jnp.sqrt(var + 1e-5) * w + b


def _l2n(h):
    nrm = jnp.sqrt(jnp.sum(h * h, axis=-1, keepdims=True))
    return h / jnp.maximum(nrm, 1e-12)


def _stage_a_body(x_ref, bnw_ref, bnb_ref, fw_ref, w1_ref, b1_ref, l1w_ref,
                  l1b_ref, w2_ref, b2_ref, l2w_ref, l2b_ref, wl0_ref, wr0_ref,
                  d_ref, p0_ref, r0_ref):
    x = x_ref[...]
    mu = jnp.mean(x, axis=0, keepdims=True)
    xc = x - mu
    var = jnp.mean(xc * xc, axis=0, keepdims=True)
    h = xc * (bnw_ref[...] / jnp.sqrt(var + 1e-5)) + bnb_ref[...]
    xw = h * (1.0 + 0.5 * jax.nn.sigmoid(fw_ref[...]))
    t = jnp.dot(xw, w1_ref[...], preferred_element_type=jnp.float32) + b1_ref[...]
    t = jnp.maximum(_ln(t, l1w_ref[...], l1b_ref[...]), 0.0)
    t = jnp.dot(t, w2_ref[...], preferred_element_type=jnp.float32) + b2_ref[...]
    d_ref[...] = jnp.maximum(_ln(t, l2w_ref[...], l2b_ref[...]), 0.0)
    p = jnp.dot(xw, wl0_ref[...], preferred_element_type=jnp.float32)
    col = lax.broadcasted_iota(jnp.int32, (x.shape[0], 16), 1)
    ones_col = jnp.where(col == 0, 1.0, 0.0)
    p0_ref[...] = jnp.concatenate([p, ones_col], axis=1)
    r0_ref[...] = jnp.dot(xw, wr0_ref[...], preferred_element_type=jnp.float32)


def _stage_a(x, bnw, bnb, fw, w1, b1, l1w, l1b, w2, b2, l2w, l2b, wl0, wr0):
    n = x.shape[0]
    return pl.pallas_call(
        _stage_a_body,
        out_shape=[
            jax.ShapeDtypeStruct((n, 64), jnp.float32),
            jax.ShapeDtypeStruct((n, 80), jnp.float32),
            jax.ShapeDtypeStruct((n, 64), jnp.float32),
        ],
    )(x, bnw, bnb, fw, w1, b1, l1w, l1b, w2, b2, l2w, l2b, wl0, wr0)


def _stage_b_body(agg_ref, r0_ref, nw_ref, nb_ref, wl_ref,
                  g_ref, p_ref, inv_ref):
    n = r0_ref.shape[0]
    agg = agg_ref[0, :n] + agg_ref[1, :n]
    deg = jnp.sum(agg[:, 64:80], axis=-1, keepdims=True)
    inv = 1.0 / jnp.maximum(deg, 1.0)
    sage = agg[:, :64] * inv + r0_ref[...]
    g = jnp.maximum(_ln(_l2n(sage), nw_ref[...], nb_ref[...]), 0.0)
    g_ref[...] = g
    p_ref[...] = jnp.dot(g, wl_ref[...], preferred_element_type=jnp.float32)
    inv_ref[...] = jnp.broadcast_to(inv, (n, 8))


def _stage_b(agg0, r0, nw, nb, wl):
    n = r0.shape[0]
    s64 = jax.ShapeDtypeStruct((n, 64), jnp.float32)
    s8 = jax.ShapeDtypeStruct((n, 8), jnp.float32)
    return pl.pallas_call(
        _stage_b_body, out_shape=[s64, s64, s8],
    )(agg0, r0, nw, nb, wl)


def _stage_c_body(agg_ref, gin_ref, inv_ref, nw_ref, nb_ref,
                  wr_ref, wl_ref, g_ref, p_ref):
    n = gin_ref.shape[0]
    agg = agg_ref[0, :n] + agg_ref[1, :n]
    gin = gin_ref[...]
    rin = jnp.dot(gin, wr_ref[...], preferred_element_type=jnp.float32)
    sage = agg * inv_ref[:, 0:1] + rin
    g = jnp.maximum(_ln(_l2n(sage), nw_ref[...], nb_ref[...]), 0.0)
    g = g + 0.1 * gin
    g_ref[...] = g
    p_ref[...] = jnp.dot(g, wl_ref[...], preferred_element_type=jnp.float32)


def _stage_c(agg, gin, inv, nw, nb, wr, wl):
    n = gin.shape[0]
    s64 = jax.ShapeDtypeStruct((n, 64), jnp.float32)
    return pl.pallas_call(
        _stage_c_body, out_shape=[s64, s64],
    )(agg, gin, inv, nw, nb, wr, wl)


def _head(comb, w1, b1, lw, lb, w2, b2, w3, b3, scale, bias):
    o = jnp.dot(comb, w1, preferred_element_type=jnp.float32) + b1
    o = jnp.maximum(_ln(o, lw, lb), 0.0)
    o = jnp.maximum(jnp.dot(o, w2, preferred_element_type=jnp.float32) + b2, 0.0)
    o = jnp.dot(o, w3, preferred_element_type=jnp.float32) + b3
    return o * jnp.abs(scale) + bias


def _stage_d_body(agg_ref, gin_ref, inv_ref, d_ref, nw_ref, nb_ref, wr_ref,
                  hw_refs, out_ref):
    n = gin_ref.shape[0]
    agg = agg_ref[0, :n] + agg_ref[1, :n]
    gin = gin_ref[...]
    rin = jnp.dot(gin, wr_ref[...], preferred_element_type=jnp.float32)
    sage = agg * inv_ref[:, 0:1] + rin
    g = jnp.maximum(_ln(_l2n(sage), nw_ref[...], nb_ref[...]), 0.0)
    g = g + 0.1 * gin
    comb = jnp.concatenate([d_ref[...], g], axis=1)
    (rw1, rb1, rlw, rlb, rw2, rb2, rw3, rb3, rs, rbi,
     ew1, eb1, elw, elb, ew2, eb2, ew3, eb3, es, ebi) = [h[...] for h in hw_refs]
    rtt = _head(comb, rw1, rb1, rlw, rlb, rw2, rb2, rw3, rb3, rs, rbi)
    ret = _head(comb, ew1, eb1, elw, elb, ew2, eb2, ew3, eb3, es, ebi)
    col = lax.broadcasted_iota(jnp.int32, (comb.shape[0], 8), 1)
    out_ref[...] = jnp.where(col == 0, rtt, jnp.where(col == 1, ret, 0.0))


def _stage_d(agg, gin, inv, d, nw, nb, wr, head_ws):
    n = gin.shape[0]

    def body(agg_ref, gin_ref, inv_ref, d_ref, nw_ref, nb_ref, wr_ref, *hw):
        _stage_d_body(agg_ref, gin_ref, inv_ref, d_ref, nw_ref, nb_ref,
                      wr_ref, hw[:-1], hw[-1])

    return pl.pallas_call(
        body, out_shape=jax.ShapeDtypeStruct((n, 8), jnp.float32),
    )(agg, gin, inv, d, nw, nb, wr, *head_ws)


@functools.lru_cache(maxsize=None)
def _make_segsum(e_total, n_pad, w, ch):
    epw = e_total // _NW
    nch = epw // ch
    rps = n_pad // _NS
    mesh = plsc.VectorSubcoreMesh(core_axis_name="c", subcore_axis_name="s")

    @functools.partial(
        pl.kernel,
        out_type=jax.ShapeDtypeStruct((_NC, n_pad, w), jnp.float32),
        mesh=mesh,
        scratch_types=[
            pltpu.VMEM((nch, ch), jnp.int32),
            pltpu.VMEM((nch, ch), jnp.int32),
            pltpu.VMEM((2, ch, w), jnp.float32),
            pltpu.VMEM_SHARED((n_pad, w), jnp.float32),
            pltpu.SemaphoreType.DMA,
            pltpu.SemaphoreType.DMA,
            pltpu.SemaphoreType.DMA,
        ],
        compiler_params=pltpu.CompilerParams(use_tc_tiling_on_sc=False),
    )
    def segsum(p_hbm, src_hbm, dst_hbm, z_hbm, out_hbm,
               src_v, dst_v, rows_v, agg_sh, semg, sems, semz):
        cid = lax.axis_index("c")
        sid = lax.axis_index("s")
        wid = sid * _NC + cid
        pltpu.async_copy(z_hbm.at[pl.ds(sid * rps, rps)],
                         agg_sh.at[pl.ds(sid * rps, rps)], semz)
        pltpu.sync_copy(src_hbm.at[wid], src_v)
        pltpu.sync_copy(dst_hbm.at[wid], dst_v)

        def g_start(i, b):
            pltpu.async_copy(p_hbm.at[src_v.at[i]], rows_v.at[b], semg)

        def g_wait(i, b):
            pltpu.make_async_copy(p_hbm.at[src_v.at[i]], rows_v.at[b],
                                  semg).wait()

        def s_start(i, b):
            pltpu.async_copy(rows_v.at[b], agg_sh.at[dst_v.at[i]], sems,
                             add=True)

        def s_wait(i, b):
            pltpu.make_async_copy(rows_v.at[b], agg_sh.at[dst_v.at[i]],
                                  sems).wait()

        g_start(0, 0)
        pltpu.make_async_copy(z_hbm.at[pl.ds(sid * rps, rps)],
                              agg_sh.at[pl.ds(sid * rps, rps)], semz).wait()
        plsc.subcore_barrier()

        def body(i, carry):
            b = lax.rem(i, 2)
            g_wait(i, b)
            s_start(i, b)

            @pl.when(i > 0)
            def _():
                s_wait(i - 1, 1 - b)

            g_start(i + 1, 1 - b)
            return carry

        lax.fori_loop(0, nch - 1, body, 0)
        bl = (nch - 1) % 2
        g_wait(nch - 1, bl)
        s_start(nch - 1, bl)
        s_wait(nch - 2, 1 - bl)
        s_wait(nch - 1, bl)
        plsc.subcore_barrier()
        pltpu.sync_copy(agg_sh.at[pl.ds(sid * rps, rps)],
                        out_hbm.at[cid].at[pl.ds(sid * rps, rps)])

    return segsum


def kernel(x, edge_index, params):
    p = params
    n = x.shape[0]
    e_total = edge_index.shape[1]
    ch80, ch64 = 200, 400
    src80 = edge_index[0].reshape(_NW, e_total // _NW // ch80, ch80)
    dst80 = edge_index[1].reshape(_NW, e_total // _NW // ch80, ch80)
    src64 = edge_index[0].reshape(_NW, e_total // _NW // ch64, ch64)
    dst64 = edge_index[1].reshape(_NW, e_total // _NW // ch64, ch64)

    row = lambda v: v.reshape(1, -1)
    d, p0, r0 = _stage_a(
        x, row(p['bn_w']), row(p['bn_b']), row(p['fw']),
        p['ft_w1'].T, row(p['ft_b1']), row(p['ft_ln1_w']), row(p['ft_ln1_b']),
        p['ft_w2'].T, row(p['ft_b2']), row(p['ft_ln2_w']), row(p['ft_ln2_b']),
        p['conv0_wl'].T, p['conv0_wr'].T)

    n_pad = ((n + 8 * _NS - 1) // (8 * _NS)) * (8 * _NS)
    z80 = jnp.zeros((n_pad, 80), jnp.float32)
    z64 = jnp.zeros((n_pad, 64), jnp.float32)
    segsum80 = _make_segsum(e_total, n_pad, 80, ch80)
    segsum64 = _make_segsum(e_total, n_pad, 64, ch64)

    agg0 = segsum80(p0, src80, dst80, z80)
    g1, p1, inv = _stage_b(agg0, r0, row(p['norm0_w']), row(p['norm0_b']),
                           p['conv1_wl'].T)
    agg1 = segsum64(p1, src64, dst64, z64)
    g2, p2 = _stage_c(agg1, g1, inv, row(p['norm1_w']), row(p['norm1_b']),
                      p['conv1_wr'].T, p['conv2_wl'].T)
    agg2 = segsum64(p2, src64, dst64, z64)

    head_ws = []
    for pre in ['rtt', 'ret']:
        head_ws += [p[pre + '_w1'].T, row(p[pre + '_b1']),
                    row(p[pre + '_ln_w']), row(p[pre + '_ln_b']),
                    p[pre + '_w2'].T, row(p[pre + '_b2']),
                    p[pre + '_w3'].T, row(p[pre + '_b3']),
                    p[pre + '_scale'].reshape(1, 1), p[pre + '_bias'].reshape(1, 1)]
    o8 = _stage_d(agg2, g2, inv, d, row(p['norm2_w']), row(p['norm2_b']),
                  p['conv2_wr'].T, head_ws)
    return o8[:, :2]


# speedup vs baseline: 13.0355x; 1.0477x over previous
import functools

import jax
import jax.numpy as jnp
from jax import lax
from jax.experimental import pallas as pl
from jax.experimental.pallas import tpu as pltpu
from jax.experimental.pallas import tpu_sc as plsc

_NC = 2
_NS = 16
_NW = _NC * _NS
_CH = 200


def _ln(h, w, b):
    mu = jnp.mean(h, axis=-1, keepdims=True)
    d = h - mu
    var = jnp.mean(d * d, axis=-1, keepdims=True)
    return d / jnp.sqrt(var + 1e-5) * w + b


def _l2n(h):
    nrm = jnp.sqrt(jnp.sum(h * h, axis=-1, keepdims=True))
    return h / jnp.maximum(nrm, 1e-12)


def _stage_a_body(x_ref, bnw_ref, bnb_ref, fw_ref, w1_ref, b1_ref, l1w_ref,
                  l1b_ref, w2_ref, b2_ref, l2w_ref, l2b_ref, wl0_ref, wr0_ref,
                  d_ref, p0_ref, r0_ref):
    x = x_ref[...]
    mu = jnp.mean(x, axis=0, keepdims=True)
    xc = x - mu
    var = jnp.mean(xc * xc, axis=0, keepdims=True)
    h = xc * (bnw_ref[...] / jnp.sqrt(var + 1e-5)) + bnb_ref[...]
    xw = h * (1.0 + 0.5 * jax.nn.sigmoid(fw_ref[...]))
    t = jnp.dot(xw, w1_ref[...], preferred_element_type=jnp.float32) + b1_ref[...]
    t = jnp.maximum(_ln(t, l1w_ref[...], l1b_ref[...]), 0.0)
    t = jnp.dot(t, w2_ref[...], preferred_element_type=jnp.float32) + b2_ref[...]
    d_ref[...] = jnp.maximum(_ln(t, l2w_ref[...], l2b_ref[...]), 0.0)
    p0_ref[...] = jnp.dot(xw, wl0_ref[...], preferred_element_type=jnp.float32)
    r0_ref[...] = jnp.dot(xw, wr0_ref[...], preferred_element_type=jnp.float32)


def _stage_a(x, bnw, bnb, fw, w1, b1, l1w, l1b, w2, b2, l2w, l2b, wl0, wr0):
    n = x.shape[0]
    return pl.pallas_call(
        _stage_a_body,
        out_shape=[
            jax.ShapeDtypeStruct((n, 64), jnp.float32),
            jax.ShapeDtypeStruct((n, 64), jnp.float32),
            jax.ShapeDtypeStruct((n, 64), jnp.float32),
        ],
    )(x, bnw, bnb, fw, w1, b1, l1w, l1b, w2, b2, l2w, l2b, wl0, wr0)


def _stage_b_body(agg_ref, deg_ref, r0_ref, nw_ref, nb_ref, wl_ref,
                  g_ref, p_ref, inv_ref):
    n = r0_ref.shape[0]
    agg = agg_ref[0, :n] + agg_ref[1, :n]
    deg = jnp.sum(deg_ref[0, :n] + deg_ref[1, :n], axis=-1, keepdims=True)
    inv = 1.0 / jnp.maximum(deg, 1.0)
    sage = agg * inv + r0_ref[...]
    g = jnp.maximum(_ln(_l2n(sage), nw_ref[...], nb_ref[...]), 0.0)
    g_ref[...] = g
    p_ref[...] = jnp.dot(g, wl_ref[...], preferred_element_type=jnp.float32)
    inv_ref[...] = jnp.broadcast_to(inv, (n, 8))


def _stage_b(agg0, deg0, r0, nw, nb, wl):
    n = r0.shape[0]
    s64 = jax.ShapeDtypeStruct((n, 64), jnp.float32)
    s8 = jax.ShapeDtypeStruct((n, 8), jnp.float32)
    return pl.pallas_call(
        _stage_b_body, out_shape=[s64, s64, s8],
    )(agg0, deg0, r0, nw, nb, wl)


def _stage_c_body(agg_ref, gin_ref, inv_ref, nw_ref, nb_ref,
                  wr_ref, wl_ref, g_ref, p_ref):
    n = gin_ref.shape[0]
    agg = agg_ref[0, :n] + agg_ref[1, :n]
    gin = gin_ref[...]
    rin = jnp.dot(gin, wr_ref[...], preferred_element_type=jnp.float32)
    sage = agg * inv_ref[:, 0:1] + rin
    g = jnp.maximum(_ln(_l2n(sage), nw_ref[...], nb_ref[...]), 0.0)
    g = g + 0.1 * gin
    g_ref[...] = g
    p_ref[...] = jnp.dot(g, wl_ref[...], preferred_element_type=jnp.float32)


def _stage_c(agg, gin, inv, nw, nb, wr, wl):
    n = gin.shape[0]
    s64 = jax.ShapeDtypeStruct((n, 64), jnp.float32)
    return pl.pallas_call(
        _stage_c_body, out_shape=[s64, s64],
    )(agg, gin, inv, nw, nb, wr, wl)


def _head(comb, w1, b1, lw, lb, w2, b2, w3, b3, scale, bias):
    o = jnp.dot(comb, w1, preferred_element_type=jnp.float32) + b1
    o = jnp.maximum(_ln(o, lw, lb), 0.0)
    o = jnp.maximum(jnp.dot(o, w2, preferred_element_type=jnp.float32) + b2, 0.0)
    o = jnp.dot(o, w3, preferred_element_type=jnp.float32) + b3
    return o * jnp.abs(scale) + bias


def _stage_d_body(agg_ref, gin_ref, inv_ref, d_ref, nw_ref, nb_ref, wr_ref,
                  hw_refs, out_ref):
    n = gin_ref.shape[0]
    agg = agg_ref[0, :n] + agg_ref[1, :n]
    gin = gin_ref[...]
    rin = jnp.dot(gin, wr_ref[...], preferred_element_type=jnp.float32)
    sage = agg * inv_ref[:, 0:1] + rin
    g = jnp.maximum(_ln(_l2n(sage), nw_ref[...], nb_ref[...]), 0.0)
    g = g + 0.1 * gin
    comb = jnp.concatenate([d_ref[...], g], axis=1)
    (rw1, rb1, rlw, rlb, rw2, rb2, rw3, rb3, rs, rbi,
     ew1, eb1, elw, elb, ew2, eb2, ew3, eb3, es, ebi) = [h[...] for h in hw_refs]
    rtt = _head(comb, rw1, rb1, rlw, rlb, rw2, rb2, rw3, rb3, rs, rbi)
    ret = _head(comb, ew1, eb1, elw, elb, ew2, eb2, ew3, eb3, es, ebi)
    col = lax.broadcasted_iota(jnp.int32, (comb.shape[0], 8), 1)
    out_ref[...] = jnp.where(col == 0, rtt, jnp.where(col == 1, ret, 0.0))


def _stage_d(agg, gin, inv, d, nw, nb, wr, head_ws):
    n = gin.shape[0]

    def body(agg_ref, gin_ref, inv_ref, d_ref, nw_ref, nb_ref, wr_ref, *hw):
        _stage_d_body(agg_ref, gin_ref, inv_ref, d_ref, nw_ref, nb_ref,
                      wr_ref, hw[:-1], hw[-1])

    return pl.pallas_call(
        body, out_shape=jax.ShapeDtypeStruct((n, 8), jnp.float32),
    )(agg, gin, inv, d, nw, nb, wr, *head_ws)


@functools.lru_cache(maxsize=None)
def _make_segsum(e_total, n_pad, w, ch, with_deg=False):
    epw = e_total // _NW
    nch = epw // ch
    rps = n_pad // _NS
    mesh = plsc.VectorSubcoreMesh(core_axis_name="c", subcore_axis_name="s")

    agg_ty = jax.ShapeDtypeStruct((_NC, n_pad, w), jnp.float32)
    out_ty = [agg_ty, jax.ShapeDtypeStruct((_NC, n_pad, 16), jnp.float32)
              ] if with_deg else agg_ty
    scratch = [
        pltpu.VMEM((nch, ch), jnp.int32),
        pltpu.VMEM((nch, ch), jnp.int32),
        pltpu.VMEM((2, ch, w), jnp.float32),
        pltpu.VMEM_SHARED((n_pad, w), jnp.float32),
        pltpu.SemaphoreType.DMA,
        pltpu.SemaphoreType.DMA,
        pltpu.SemaphoreType.DMA,
    ]
    if with_deg:
        scratch += [
            pltpu.VMEM((ch, 16), jnp.float32),
            pltpu.VMEM_SHARED((n_pad, 16), jnp.float32),
            pltpu.SemaphoreType.DMA,
        ]

    @functools.partial(
        pl.kernel, out_type=out_ty, mesh=mesh, scratch_types=scratch,
        compiler_params=pltpu.CompilerParams(use_tc_tiling_on_sc=False),
    )
    def segsum(p_hbm, src_hbm, dst_hbm, z_hbm, *rest):
        if with_deg:
            (ones_hbm, out_hbm, deg_hbm, src_v, dst_v, rows_v, agg_sh,
             semg, sems, semz, ones_v, deg_sh, semd) = rest
        else:
            (out_hbm, src_v, dst_v, rows_v, agg_sh, semg, sems, semz) = rest
        cid = lax.axis_index("c")
        sid = lax.axis_index("s")
        wid = sid * _NC + cid
        pltpu.async_copy(z_hbm.at[pl.ds(sid * rps, rps)],
                         agg_sh.at[pl.ds(sid * rps, rps)], semz)
        if with_deg:
            pltpu.async_copy(z_hbm.at[pl.ds(sid * rps, rps), pl.ds(0, 16)],
                             deg_sh.at[pl.ds(sid * rps, rps)], semz)
            pltpu.sync_copy(ones_hbm, ones_v)
        pltpu.sync_copy(src_hbm.at[wid], src_v)
        pltpu.sync_copy(dst_hbm.at[wid], dst_v)

        def g_start(i, b):
            pltpu.async_copy(p_hbm.at[src_v.at[i]], rows_v.at[b], semg)

        def g_wait(i, b):
            pltpu.make_async_copy(p_hbm.at[src_v.at[i]], rows_v.at[b],
                                  semg).wait()

        def s_start(i, b):
            pltpu.async_copy(rows_v.at[b], agg_sh.at[dst_v.at[i]], sems,
                             add=True)
            if with_deg:
                pltpu.async_copy(ones_v, deg_sh.at[dst_v.at[i]], semd,
                                 add=True)

        def s_wait(i, b):
            pltpu.make_async_copy(rows_v.at[b], agg_sh.at[dst_v.at[i]],
                                  sems).wait()

        g_start(0, 0)
        pltpu.make_async_copy(z_hbm.at[pl.ds(sid * rps, rps)],
                              agg_sh.at[pl.ds(sid * rps, rps)], semz).wait()
        if with_deg:
            pltpu.make_async_copy(
                z_hbm.at[pl.ds(sid * rps, rps), pl.ds(0, 16)],
                deg_sh.at[pl.ds(sid * rps, rps)], semz).wait()
        plsc.subcore_barrier()

        def body(i, carry):
            b = lax.rem(i, 2)
            g_wait(i, b)
            s_start(i, b)

            @pl.when(i > 0)
            def _():
                s_wait(i - 1, 1 - b)

            g_start(i + 1, 1 - b)
            return carry

        lax.fori_loop(0, nch - 1, body, 0)
        bl = (nch - 1) % 2
        g_wait(nch - 1, bl)
        s_start(nch - 1, bl)
        s_wait(nch - 2, 1 - bl)
        s_wait(nch - 1, bl)
        if with_deg:
            def drain(i, carry):
                pltpu.make_async_copy(ones_v, deg_sh.at[dst_v.at[0]],
                                      semd).wait()
                return carry
            lax.fori_loop(0, nch, drain, 0)
        plsc.subcore_barrier()
        pltpu.sync_copy(agg_sh.at[pl.ds(sid * rps, rps)],
                        out_hbm.at[cid].at[pl.ds(sid * rps, rps)])
        if with_deg:
            pltpu.sync_copy(deg_sh.at[pl.ds(sid * rps, rps)],
                            deg_hbm.at[cid].at[pl.ds(sid * rps, rps)])

    return segsum


def kernel(x, edge_index, params):
    p = params
    n = x.shape[0]
    e_total = edge_index.shape[1]
    ch = 400
    src3 = edge_index[0].reshape(_NW, e_total // _NW // ch, ch)
    dst3 = edge_index[1].reshape(_NW, e_total // _NW // ch, ch)

    row = lambda v: v.reshape(1, -1)
    d, p0, r0 = _stage_a(
        x, row(p['bn_w']), row(p['bn_b']), row(p['fw']),
        p['ft_w1'].T, row(p['ft_b1']), row(p['ft_ln1_w']), row(p['ft_ln1_b']),
        p['ft_w2'].T, row(p['ft_b2']), row(p['ft_ln2_w']), row(p['ft_ln2_b']),
        p['conv0_wl'].T, p['conv0_wr'].T)

    n_pad = ((n + 8 * _NS - 1) // (8 * _NS)) * (8 * _NS)
    z64 = jnp.zeros((n_pad, 64), jnp.float32)
    col = jnp.arange(16)[None, :]
    ones16 = jnp.where(col == 0, 1.0, 0.0) * jnp.ones((ch, 1), jnp.float32)
    segsum_deg = _make_segsum(e_total, n_pad, 64, ch, with_deg=True)
    segsum64 = _make_segsum(e_total, n_pad, 64, ch)

    agg0, deg0 = segsum_deg(p0, src3, dst3, z64, ones16)
    g1, p1, inv = _stage_b(agg0, deg0, r0, row(p['norm0_w']),
                           row(p['norm0_b']), p['conv1_wl'].T)
    agg1 = segsum64(p1, src3, dst3, z64)
    g2, p2 = _stage_c(agg1, g1, inv, row(p['norm1_w']), row(p['norm1_b']),
                      p['conv1_wr'].T, p['conv2_wl'].T)
    agg2 = segsum64(p2, src3, dst3, z64)

    head_ws = []
    for pre in ['rtt', 'ret']:
        head_ws += [p[pre + '_w1'].T, row(p[pre + '_b1']),
                    row(p[pre + '_ln_w']), row(p[pre + '_ln_b']),
                    p[pre + '_w2'].T, row(p[pre + '_b2']),
                    p[pre + '_w3'].T, row(p[pre + '_b3']),
                    p[pre + '_scale'].reshape(1, 1), p[pre + '_bias'].reshape(1, 1)]
    o8 = _stage_d(agg2, g2, inv, d, row(p['norm2_w']), row(p['norm2_b']),
                  p['conv2_wr'].T, head_ws)
    return o8[:, :2]
